# baseline (speedup 1.0000x reference)
import functools

import jax
import jax.numpy as jnp
from jax import lax
from jax.experimental import pallas as pl
from jax.experimental.pallas import tpu as pltpu
from jax.experimental.pallas import tpu_sc as plsc

_N = 4096
_INF = float("inf")
_NW = 32
_RPW = _N // _NW
_L = 16


def _sc_sel(v, i):
    lane = lax.iota(jnp.int32, _L)
    return jnp.min(jnp.where(lane == i, v, _INF))


def _sc_row_topk(qx, qy, qz, qsq, cx, cy, cz, csq):
    def chunk(ci, carry):
        top16, thr = carry
        sl = pl.ds(ci * _L, _L)
        cross = (qx * cx[sl] + qy * cy[sl]) + qz * cz[sl]
        dv = (-2.0 * cross + qsq) + csq[sl]

        def merge(op):
            t16, _ = op
            sd = lax.sort(dv, dimension=0)
            lo = jnp.minimum(t16, lax.rev(sd, dimensions=(0,)))
            new16 = lax.sort(lo, dimension=0)
            return new16, jnp.max(new16)

        return lax.cond(jnp.any(dv < thr), merge, lambda op: op, (top16, thr))

    top16 = jnp.full((_L,), _INF, jnp.float32)
    top16, _ = lax.fori_loop(0, _N // _L, chunk, (top16, jnp.float32(_INF)))
    return top16


def _sc_matrix_pass(wid, qbufs, cbufs, kpairs, thbufs, outs):
    qx, qy, qz, qsq = qbufs
    cx, cy, cz, csq = cbufs
    lane = lax.iota(jnp.int32, _L)

    def group(g, _):
        base = wid * _RPW + g * _L
        qxv = qx[pl.ds(base, _L)]
        qyv = qy[pl.ds(base, _L)]
        qzv = qz[pl.ds(base, _L)]
        qsv = qsq[pl.ds(base, _L)]

        def row(j, accs):
            top16 = _sc_row_topk(_sc_sel(qxv, j), _sc_sel(qyv, j),
                                 _sc_sel(qzv, j), _sc_sel(qsv, j),
                                 cx, cy, cz, csq)
            new = []
            for (ka, kb), acc in zip(kpairs, accs):
                mid = 0.5 * (_sc_sel(top16, ka) + _sc_sel(top16, kb))
                new.append(jnp.where(lane == j, mid, acc))
            return tuple(new)

        accs = tuple(jnp.zeros((_L,), jnp.float32) for _ in kpairs)
        accs = lax.fori_loop(0, _L, row, accs)
        for buf, acc in zip(thbufs, accs):
            buf[pl.ds(g * _L, _L)] = acc
        return 0

    lax.fori_loop(0, _RPW // _L, group, 0)
    for buf, out in zip(thbufs, outs):
        pltpu.sync_copy(buf, out.at[pl.ds(wid * _RPW, _RPW)])


def _sc_body(p1x_hbm, p1y_hbm, p1z_hbm, p2x_hbm, p2y_hbm, p2z_hbm,
             wx_hbm, wy_hbm, wz_hbm, p1sq_hbm, p2sq_hbm, wsq_hbm,
             th22_hbm, th11a_hbm, th11b_hbm, th12_hbm,
             b1x, b1y, b1z, s1, b2x, b2y, b2z, s2, bwx, bwy, bwz, sw,
             t22, t11a, t11b, t12):
    wid = lax.axis_index("s") * 2 + lax.axis_index("c")
    for src, dst in ((p1x_hbm, b1x), (p1y_hbm, b1y), (p1z_hbm, b1z),
                     (p2x_hbm, b2x), (p2y_hbm, b2y), (p2z_hbm, b2z),
                     (wx_hbm, bwx), (wy_hbm, bwy), (wz_hbm, bwz),
                     (p1sq_hbm, s1), (p2sq_hbm, s2), (wsq_hbm, sw)):
        pltpu.sync_copy(src, dst)

    pc1 = (b1x, b1y, b1z, s1)
    pc2 = (b2x, b2y, b2z, s2)
    wrp = (bwx, bwy, bwz, sw)
    _sc_matrix_pass(wid, pc2, pc2, [(9, 10)], [t22], [th22_hbm])
    _sc_matrix_pass(wid, pc1, pc1, [(9, 10), (8, 9)], [t11a, t11b],
                    [th11a_hbm, th11b_hbm])
    _sc_matrix_pass(wid, wrp, pc2, [(4, 5)], [t12], [th12_hbm])


def _sc_thresholds(p1r, p2r, wr, p1sq, p2sq, wsq):
    mesh = plsc.VectorSubcoreMesh(core_axis_name="c", subcore_axis_name="s")
    f32 = jnp.float32
    kern = functools.partial(
        pl.kernel,
        mesh=mesh,
        out_type=[jax.ShapeDtypeStruct((_N,), f32) for _ in range(4)],
        scratch_types=(
            [pltpu.VMEM((_N,), f32) for _ in range(12)]
            + [pltpu.VMEM((_RPW,), f32) for _ in range(4)]
        ),
        compiler_params=pltpu.CompilerParams(needs_layout_passes=False),
    )(_sc_body)
    return kern(p1r[0], p1r[1], p1r[2], p2r[0], p2r[1], p2r[2],
                wr[0], wr[1], wr[2], p1sq, p2sq, wsq)


def _sqdist(rows_mat, cols_mat, query_is_col):
    cross = jnp.dot(rows_mat, cols_mat, preferred_element_type=jnp.float32)
    rowsq = (rows_mat[:, 0:1] * rows_mat[:, 0:1]
             + rows_mat[:, 1:2] * rows_mat[:, 1:2]
             + rows_mat[:, 2:3] * rows_mat[:, 2:3])
    colsq = (cols_mat[0:1, :] * cols_mat[0:1, :]
             + cols_mat[1:2, :] * cols_mat[1:2, :]
             + cols_mat[2:3, :] * cols_mat[2:3, :])
    if query_is_col:
        return (-2.0 * cross + colsq) + rowsq
    return (-2.0 * cross + rowsq) + colsq


def _curv2_body(c_ref, g_ref, ct_ref, gt_ref, th_ref, out_ref):
    rows_mat = c_ref[...] + g_ref[...]
    cols_mat = ct_ref[...] + gt_ref[...]
    d2 = _sqdist(rows_mat, cols_mat, query_is_col=True)
    mask = d2 <= th_ref[...]
    for d in range(3):
        s = jnp.sum(jnp.where(mask, rows_mat[:, d : d + 1], 0.0),
                    axis=0, keepdims=True)
        out_ref[d : d + 1, :] = ((s - 10.0 * cols_mat[d : d + 1, :])
                                 * jnp.float32(1.0 / 9.0))


def _main_body(c_ref, g_ref, f_ref, tha_ref, thb_ref, thc_ref,
               ct_ref, gt_ref, ft_ref, cv_ref, out_ref,
               colmin_ref, acc_ref, *, tiles):
    i = pl.program_id(0)

    @pl.when(i == 0)
    def _init():
        acc_ref[0] = 0.0
        acc_ref[1] = 0.0
        acc_ref[2] = 0.0
        colmin_ref[...] = jnp.full(colmin_ref.shape, _INF, jnp.float32)

    c_row = c_ref[...]
    f_row = f_ref[...]
    w_row = c_row + f_row
    ct_all = ct_ref[...]
    gt_all = gt_ref[...]
    ft_all = ft_ref[...]
    p2cols = ct_all + gt_all
    wcols = ct_all + ft_all

    fr = [f_row[:, d : d + 1] for d in range(3)]
    wr = [w_row[:, d : d + 1] for d in range(3)]
    fc = [ft_all[d : d + 1, :] for d in range(3)]
    wc = [wcols[d : d + 1, :] for d in range(3)]
    cv = [cv_ref[d : d + 1, :] for d in range(3)]

    d11 = _sqdist(c_row, ct_all, query_is_col=False)
    mask10 = d11 <= tha_ref[...]
    mask9 = d11 <= thb_ref[...]
    moved = []
    for d in range(3):
        s = jnp.sum(jnp.where(mask10, wc[d], 0.0), axis=1, keepdims=True)
        moved.append((s - 10.0 * wr[d]) * jnp.float32(1.0 / 9.0))
    nrm = jnp.sqrt((fc[0] - fr[0]) ** 2 + (fc[1] - fr[1]) ** 2
                   + (fc[2] - fr[2]) ** 2)
    smooth_part = jnp.sum(jnp.where(mask9, nrm, 0.0)) * jnp.float32(1.0 / 8.0)

    d12 = _sqdist(w_row, p2cols, query_is_col=False)
    d1 = jnp.min(d12, axis=1, keepdims=True)
    cham1_part = jnp.sum(d1)
    colmin_ref[...] = jnp.minimum(colmin_ref[...],
                                  jnp.min(d12, axis=0, keepdims=True))
    w = jnp.where(d12 <= thc_ref[...], 1.0 / (d12 + 1e-8), 0.0)
    wnorm = jnp.sum(w, axis=1, keepdims=True)
    curv_part = jnp.float32(0.0)
    for d in range(3):
        inter = jnp.sum(w * cv[d], axis=1, keepdims=True) / wnorm
        curv_part = curv_part + jnp.sum((inter - moved[d]) ** 2)

    acc_ref[0] += cham1_part
    acc_ref[1] += smooth_part
    acc_ref[2] += curv_part

    @pl.when(i == tiles - 1)
    def _fin():
        cham2 = jnp.sum(colmin_ref[...])
        total = (jnp.float32(0.02) * (acc_ref[0] + cham2)
                 + jnp.float32(0.02) * acc_ref[1]
                 + jnp.float32(0.006) * acc_ref[2])
        out_ref[...] = total[None, None]


def kernel(registration_pred, registration_gt, coords):
    c = coords
    g = registration_gt[0]
    f = registration_pred[0]
    ct = c.T
    gt_ = g.T
    ft = f.T
    p2 = c + g
    wp = c + f

    rp = lambda a: lax.reduce_precision(a, 8, 7)
    p1r = rp(ct)
    p2r = rp(p2.T)
    wr = rp(wp.T)
    p1sq = jnp.sum(c ** 2, -1)
    p2sq = jnp.sum(p2 ** 2, -1)
    wsq = jnp.sum(wp ** 2, -1)

    th22, th11a, th11b, th12 = _sc_thresholds(p1r, p2r, wr, p1sq, p2sq, wsq)

    ta = 512
    curv2_t = pl.pallas_call(
        _curv2_body,
        grid=(_N // ta,),
        in_specs=[
            pl.BlockSpec((_N, 3), lambda i: (0, 0)),
            pl.BlockSpec((_N, 3), lambda i: (0, 0)),
            pl.BlockSpec((3, ta), lambda i: (0, i)),
            pl.BlockSpec((3, ta), lambda i: (0, i)),
            pl.BlockSpec((1, ta), lambda i: (0, i)),
        ],
        out_specs=pl.BlockSpec((3, ta), lambda i: (0, i)),
        out_shape=jax.ShapeDtypeStruct((3, _N), jnp.float32),
    )(c, g, ct, gt_, th22.reshape(1, _N))

    tb = 256
    tiles = _N // tb
    total = pl.pallas_call(
        functools.partial(_main_body, tiles=tiles),
        grid=(tiles,),
        in_specs=[
            pl.BlockSpec((tb, 3), lambda i: (i, 0)),
            pl.BlockSpec((tb, 3), lambda i: (i, 0)),
            pl.BlockSpec((tb, 3), lambda i: (i, 0)),
            pl.BlockSpec((tb, 1), lambda i: (i, 0)),
            pl.BlockSpec((tb, 1), lambda i: (i, 0)),
            pl.BlockSpec((tb, 1), lambda i: (i, 0)),
            pl.BlockSpec((3, _N), lambda i: (0, 0)),
            pl.BlockSpec((3, _N), lambda i: (0, 0)),
            pl.BlockSpec((3, _N), lambda i: (0, 0)),
            pl.BlockSpec((3, _N), lambda i: (0, 0)),
        ],
        out_specs=pl.BlockSpec((1, 1), lambda i: (0, 0)),
        out_shape=jax.ShapeDtypeStruct((1, 1), jnp.float32),
        scratch_shapes=[
            pltpu.VMEM((1, _N), jnp.float32),
            pltpu.SMEM((4,), jnp.float32),
        ],
    )(c, g, f, th11a.reshape(_N, 1), th11b.reshape(_N, 1),
      th12.reshape(_N, 1), ct, gt_, ft, curv2_t)

    return total.reshape(1)


# speedup vs baseline: 15.4336x; 15.4336x over previous
import functools

import jax
import jax.numpy as jnp
from jax.experimental import pallas as pl
from jax.experimental.pallas import tpu as pltpu

_N = 4096
_INF = float("inf")


def _kth_smallest(d, k, axis):
    m = jnp.min(d, axis=axis, keepdims=True)
    prev = m
    for _ in range(k - 1):
        prev = m
        m = jnp.min(jnp.where(d > m, d, _INF), axis=axis, keepdims=True)
    return m, prev


def _sqdist(rows_mat, cols_mat, query_is_col):
    cross = jnp.dot(rows_mat, cols_mat, preferred_element_type=jnp.float32)
    rowsq = (rows_mat[:, 0:1] * rows_mat[:, 0:1]
             + rows_mat[:, 1:2] * rows_mat[:, 1:2]
             + rows_mat[:, 2:3] * rows_mat[:, 2:3])
    colsq = (cols_mat[0:1, :] * cols_mat[0:1, :]
             + cols_mat[1:2, :] * cols_mat[1:2, :]
             + cols_mat[2:3, :] * cols_mat[2:3, :])
    if query_is_col:
        return (-2.0 * cross + colsq) + rowsq
    return (-2.0 * cross + rowsq) + colsq


def _curv2_body(c_ref, g_ref, ct_ref, gt_ref, out_ref):
    rows_mat = c_ref[...] + g_ref[...]
    cols_mat = ct_ref[...] + gt_ref[...]
    colp = [cols_mat[d : d + 1, :] for d in range(3)]
    rowp = [rows_mat[:, d : d + 1] for d in range(3)]
    d2 = _sqdist(rows_mat, cols_mat, query_is_col=True)
    m10, _ = _kth_smallest(d2, 10, axis=0)
    mask = d2 <= m10
    for d in range(3):
        s = jnp.sum(jnp.where(mask, rowp[d], 0.0), axis=0, keepdims=True)
        out_ref[d : d + 1, :] = (s - 10.0 * colp[d]) * jnp.float32(1.0 / 9.0)


def _main_body(c_ref, g_ref, f_ref, ct_ref, gt_ref, ft_ref, cv_ref, out_ref,
               colmin_ref, acc_ref, *, tiles):
    i = pl.program_id(0)

    @pl.when(i == 0)
    def _init():
        acc_ref[0] = 0.0
        acc_ref[1] = 0.0
        acc_ref[2] = 0.0
        colmin_ref[...] = jnp.full(colmin_ref.shape, _INF, jnp.float32)

    c_row = c_ref[...]
    f_row = f_ref[...]
    w_row = c_row + f_row
    ct_all = ct_ref[...]
    gt_all = gt_ref[...]
    ft_all = ft_ref[...]
    p2cols = ct_all + gt_all
    wcols = ct_all + ft_all

    fr = [f_row[:, d : d + 1] for d in range(3)]
    wr = [w_row[:, d : d + 1] for d in range(3)]
    fc = [ft_all[d : d + 1, :] for d in range(3)]
    wc = [wcols[d : d + 1, :] for d in range(3)]
    cv = [cv_ref[d : d + 1, :] for d in range(3)]

    d11 = _sqdist(c_row, ct_all, query_is_col=False)
    m10, m9 = _kth_smallest(d11, 10, axis=1)
    mask10 = d11 <= m10
    mask9 = d11 <= m9
    moved = []
    for d in range(3):
        s = jnp.sum(jnp.where(mask10, wc[d], 0.0), axis=1, keepdims=True)
        moved.append((s - 10.0 * wr[d]) * jnp.float32(1.0 / 9.0))
    nrm = jnp.sqrt((fc[0] - fr[0]) ** 2 + (fc[1] - fr[1]) ** 2
                   + (fc[2] - fr[2]) ** 2)
    smooth_part = jnp.sum(jnp.where(mask9, nrm, 0.0)) * jnp.float32(1.0 / 8.0)

    d12 = _sqdist(w_row, p2cols, query_is_col=False)
    d1 = jnp.min(d12, axis=1, keepdims=True)
    cham1_part = jnp.sum(d1)
    colmin_ref[...] = jnp.minimum(colmin_ref[...],
                                  jnp.min(d12, axis=0, keepdims=True))
    m5 = d1
    for _ in range(4):
        m5 = jnp.min(jnp.where(d12 > m5, d12, _INF), axis=1, keepdims=True)
    w = jnp.where(d12 <= m5, 1.0 / (d12 + 1e-8), 0.0)
    wnorm = jnp.sum(w, axis=1, keepdims=True)
    curv_part = jnp.float32(0.0)
    for d in range(3):
        inter = jnp.sum(w * cv[d], axis=1, keepdims=True) / wnorm
        curv_part = curv_part + jnp.sum((inter - moved[d]) ** 2)

    acc_ref[0] += cham1_part
    acc_ref[1] += smooth_part
    acc_ref[2] += curv_part

    @pl.when(i == tiles - 1)
    def _fin():
        cham2 = jnp.sum(colmin_ref[...])
        total = (jnp.float32(0.02) * (acc_ref[0] + cham2)
                 + jnp.float32(0.02) * acc_ref[1]
                 + jnp.float32(0.006) * acc_ref[2])
        out_ref[...] = total[None, None]


def kernel(registration_pred, registration_gt, coords):
    c = coords
    g = registration_gt[0]
    f = registration_pred[0]
    ct = c.T
    gt_ = g.T
    ft = f.T

    ta = 512
    curv2_t = pl.pallas_call(
        _curv2_body,
        grid=(_N // ta,),
        in_specs=[
            pl.BlockSpec((_N, 3), lambda i: (0, 0)),
            pl.BlockSpec((_N, 3), lambda i: (0, 0)),
            pl.BlockSpec((3, ta), lambda i: (0, i)),
            pl.BlockSpec((3, ta), lambda i: (0, i)),
        ],
        out_specs=pl.BlockSpec((3, ta), lambda i: (0, i)),
        out_shape=jax.ShapeDtypeStruct((3, _N), jnp.float32),
    )(c, g, ct, gt_)

    tb = 256
    tiles = _N // tb
    total = pl.pallas_call(
        functools.partial(_main_body, tiles=tiles),
        grid=(tiles,),
        in_specs=[
            pl.BlockSpec((tb, 3), lambda i: (i, 0)),
            pl.BlockSpec((tb, 3), lambda i: (i, 0)),
            pl.BlockSpec((tb, 3), lambda i: (i, 0)),
            pl.BlockSpec((3, _N), lambda i: (0, 0)),
            pl.BlockSpec((3, _N), lambda i: (0, 0)),
            pl.BlockSpec((3, _N), lambda i: (0, 0)),
            pl.BlockSpec((3, _N), lambda i: (0, 0)),
        ],
        out_specs=pl.BlockSpec((1, 1), lambda i: (0, 0)),
        out_shape=jax.ShapeDtypeStruct((1, 1), jnp.float32),
        scratch_shapes=[
            pltpu.VMEM((1, _N), jnp.float32),
            pltpu.SMEM((4,), jnp.float32),
        ],
    )(c, g, f, ct, gt_, ft, curv2_t)

    return total.reshape(1)
